# baseline (speedup 1.0000x reference)
import functools

import jax
import jax.numpy as jnp
from jax import lax
from jax.experimental import pallas as pl
from jax.experimental.pallas import tpu as pltpu
from jax.experimental.pallas import tpu_sc as plsc

N = 10000
NPAD = 10240
E = 320000
F = 128
H = 128
B = 2
T = 8
K = B * T
ET = E + N
TR = 164
EPAD = TR * 128 * 16
DEG_CHUNK = EPAD // 32
NB = 5
NBK = N // NB
GARBAGE_ROW = N
AGGR = 10016
PAIRS = K // 2

_sc_mesh = plsc.VectorSubcoreMesh(
    core_axis_name="c", subcore_axis_name="s", num_cores=2, num_subcores=16)


@functools.partial(
    pl.kernel,
    out_type=jax.ShapeDtypeStruct((32, NPAD), jnp.float32),
    mesh=_sc_mesh,
    compiler_params=pltpu.CompilerParams(needs_layout_passes=False, use_tc_tiling_on_sc=False),
    scratch_types=[
        pltpu.VMEM((DEG_CHUNK,), jnp.int32),
        pltpu.VMEM((NPAD,), jnp.float32),
    ])
def _deg(dstv_hbm, out_hbm, dloc, degv):
    c = lax.axis_index("c")
    s = lax.axis_index("s")
    wid = s * 2 + c
    pltpu.sync_copy(dstv_hbm.at[pl.ds(wid * DEG_CHUNK, DEG_CHUNK)], dloc)

    def zbody(i, carry):
        degv[pl.ds(i * 16, 16)] = jnp.zeros((16,), jnp.float32)
        return carry

    lax.fori_loop(0, NPAD // 16, zbody, 0)
    ones = jnp.ones((16,), jnp.float32)

    def sbody(i, carry):
        idx = dloc[pl.ds(i * 16, 16)]
        plsc.addupdate_scatter(degv, [idx], ones)
        return carry

    lax.fori_loop(0, DEG_CHUNK // 16, sbody, 0)
    pltpu.sync_copy(degv, out_hbm.at[wid])


def _dinv_body(parts_ref, out_ref):
    parts = parts_ref[...]
    ones = jnp.ones((32, 1), jnp.float32)
    tot = lax.dot_general(parts, ones, (((0,), (0,)), ((), ())),
                          preferred_element_type=jnp.float32)
    out_ref[...] = lax.rsqrt(tot)


def _dinv(parts):
    return pl.pallas_call(
        _dinv_body,
        out_shape=jax.ShapeDtypeStruct((NPAD, 1), jnp.float32),
    )(parts)


def _table_body(x_ref, w_ref, dinv_ref, out_ref):
    w = w_ref[...]
    d = dinv_ref[...]
    h0 = jnp.dot(x_ref[0, 0], w, preferred_element_type=jnp.float32)
    h1 = jnp.dot(x_ref[0, 1], w, preferred_element_type=jnp.float32)
    out_ref[...] = jnp.concatenate([h0 * d, h1 * d],
                                   axis=1).astype(jnp.bfloat16)


def _table(x_tab, W_gcn, dinv):
    return pl.pallas_call(
        _table_body,
        grid=(PAIRS, NB),
        in_specs=[
            pl.BlockSpec((1, 2, NBK, F), lambda p, nb: (p, 0, nb, 0)),
            pl.BlockSpec((F, H), lambda p, nb: (0, 0)),
            pl.BlockSpec((NBK, 1), lambda p, nb: (nb, 0)),
        ],
        out_specs=pl.BlockSpec((NBK, 2 * H), lambda p, nb: (p * NB + nb, 0)),
        out_shape=jax.ShapeDtypeStruct((PAIRS * N, 2 * H), jnp.bfloat16),
    )(x_tab, W_gcn, dinv)


@functools.partial(
    pl.kernel,
    out_type=jax.ShapeDtypeStruct((PAIRS, N, 2 * H), jnp.bfloat16),
    mesh=_sc_mesh,
    compiler_params=pltpu.CompilerParams(needs_layout_passes=False, use_tc_tiling_on_sc=False),
    scratch_types=(
        [pltpu.VMEM((128,), jnp.int32) for _ in range(6)]
        + [pltpu.VMEM((128,), jnp.int32) for _ in range(6)]
        + [pltpu.VMEM((128, 2 * H), jnp.bfloat16) for _ in range(3)]
        + [pltpu.VMEM_SHARED((AGGR, 2 * H), jnp.bfloat16)]
        + [pltpu.SemaphoreType.DMA for _ in range(15)]
    ))
def _seg(table_hbm, gsrc_hbm, dstv_hbm, zeros_hbm, agg_hbm,
         gb0, gb1, gb2, gb3, gb4, gb5, db0, db1, db2, db3, db4, db5,
         r0, r1, r2, aggsh,
         si0, si1, si2, si3, si4, si5, sj0, sj1, sj2, sj3, sj4, sj5,
         sg0, sg1, sg2):
    c = lax.axis_index("c")
    s = lax.axis_index("s")
    base = s * (TR * 128)
    gb = [gb0, gb1, gb2, gb3, gb4, gb5]
    db = [db0, db1, db2, db3, db4, db5]
    rr = [r0, r1, r2]
    si = [si0, si1, si2, si3, si4, si5]
    sj = [sj0, sj1, sj2, sj3, sj4, sj5]
    sg = [sg0, sg1, sg2]

    def fire_idx(b, j, pr):
        pltpu.async_copy(gsrc_hbm.at[pr, pl.ds(base + j * 128, 128)],
                         gb[b], si[b])
        pltpu.async_copy(dstv_hbm.at[pl.ds(base + j * 128, 128)],
                         db[b], sj[b])

    def wait_idx(b, pr):
        pltpu.make_async_copy(gsrc_hbm.at[pr, pl.ds(base, 128)],
                              gb[b], si[b]).wait()
        pltpu.make_async_copy(dstv_hbm.at[pl.ds(base, 128)],
                              db[b], sj[b]).wait()

    def fire_gather(b, m):
        pltpu.async_copy(table_hbm.at[gb[b]], rr[m], sg[m])

    def wait_gather(b, m):
        pltpu.make_async_copy(table_hbm.at[gb[b]], rr[m], sg[m]).wait()

    def scatter(b, m):
        pltpu.sync_copy(rr[m], aggsh.at[db[b]], add=True)

    for p in range(PAIRS // 2):
        pr = c + 2 * p
        pltpu.sync_copy(zeros_hbm, aggsh.at[pl.ds(s * (AGGR // 16),
                                                  AGGR // 16)])
        for b in range(6):
            fire_idx(b, b, pr)
        wait_idx(0, pr)
        fire_gather(0, 0)
        plsc.subcore_barrier()
        wait_gather(0, 0)
        scatter(0, 0)
        wait_idx(1, pr)
        fire_gather(1, 1)
        wait_gather(1, 1)
        scatter(1, 1)
        wait_idx(2, pr)
        fire_gather(2, 2)

        def body(i, carry):
            jo = 2 + 6 * i
            for u in range(6):
                j = jo + u
                m = (2 + u) % 3
                b = (2 + u) % 6
                m1 = (m + 1) % 3
                b1 = (b + 1) % 6
                b4 = (b + 4) % 6
                wait_gather(b, m)
                scatter(b, m)
                fire_idx(b4, j + 4, pr)
                wait_idx(b1, pr)
                fire_gather(b1, m1)
            return carry

        lax.fori_loop(0, (TR - 8) // 6, body, 0)
        for u in range(6):
            j = TR - 6 + u
            m = j % 3
            b = j % 6
            m1 = (m + 1) % 3
            b1 = (b + 1) % 6
            b4 = (b + 4) % 6
            wait_gather(b, m)
            scatter(b, m)
            if u < 2:
                fire_idx(b4, j + 4, pr)
            if u < 5:
                wait_idx(b1, pr)
                fire_gather(b1, m1)
        plsc.subcore_barrier()
        pltpu.sync_copy(aggsh.at[pl.ds(s * (N // 16), N // 16)],
                        agg_hbm.at[pr, pl.ds(s * (N // 16), N // 16)])
        plsc.subcore_barrier()


def _emb_body(agg_ref, dinv_ref, bg_ref, out_ref):
    nb = pl.program_id(1)
    blk = agg_ref[0].astype(jnp.float32)
    d = dinv_ref[...]
    bg = bg_ref[...]
    a0 = jnp.maximum(blk[:, :H] * d + bg, 0.0)
    a1 = jnp.maximum(blk[:, H:] * d + bg, 0.0)
    p0 = jnp.sum(a0, axis=0, keepdims=True)
    p1 = jnp.sum(a1, axis=0, keepdims=True)
    part = jnp.concatenate([p0, p1], axis=0).reshape(1, 2, H) * (1.0 / N)

    @pl.when(nb == 0)
    def _init():
        out_ref[...] = part

    @pl.when(nb > 0)
    def _acc():
        out_ref[...] = out_ref[...] + part


def _emb(agg, dinv, bg):
    return pl.pallas_call(
        _emb_body,
        grid=(PAIRS, NB),
        in_specs=[
            pl.BlockSpec((1, NBK, 2 * H), lambda p, nb: (p, nb, 0)),
            pl.BlockSpec((NBK, 1), lambda p, nb: (nb, 0)),
            pl.BlockSpec((1, H), lambda p, nb: (0, 0)),
        ],
        out_specs=pl.BlockSpec((1, 2, H), lambda p, nb: (p, 0, 0)),
        out_shape=jax.ShapeDtypeStruct((PAIRS, 2, H), jnp.float32),
    )(agg, dinv, bg)


def _head_body(emb_ref, wih_ref, whh_ref, bih_ref, bhh_ref, wcls_ref,
               bcls_ref, out_ref):
    wih = wih_ref[...]
    whh = whh_ref[...]
    b = bih_ref[...] + bhh_ref[...]

    def step(t, hc):
        h, cc = hc
        x_t = emb_ref[t]
        gates = (lax.dot_general(x_t, wih, (((1,), (1,)), ((), ())),
                                 preferred_element_type=jnp.float32)
                 + lax.dot_general(h, whh, (((1,), (1,)), ((), ())),
                                   preferred_element_type=jnp.float32)
                 + b)
        i = jax.nn.sigmoid(gates[:, 0 * H:1 * H])
        f = jax.nn.sigmoid(gates[:, 1 * H:2 * H])
        g = jnp.tanh(gates[:, 2 * H:3 * H])
        o = jax.nn.sigmoid(gates[:, 3 * H:4 * H])
        c_new = f * cc + i * g
        h_new = o * jnp.tanh(c_new)
        return (h_new, c_new)

    h0 = jnp.zeros((B, H), jnp.float32)
    c0 = jnp.zeros((B, H), jnp.float32)
    h, _ = lax.fori_loop(0, T, step, (h0, c0))
    out_ref[...] = jax.nn.sigmoid(
        jnp.dot(h, wcls_ref[...], preferred_element_type=jnp.float32)
        + bcls_ref[...])


def _head(emb_tbh, W_ih, W_hh, b_ih, b_hh, W_cls, b_cls):
    return pl.pallas_call(
        _head_body,
        out_shape=jax.ShapeDtypeStruct((B, 1), jnp.float32),
    )(emb_tbh, W_ih, W_hh, b_ih, b_hh, W_cls, b_cls)


def kernel(x_seq, edge_index, W_gcn, b_gcn, W_ih, W_hh, b_ih, b_hh, W_cls,
           b_cls):
    loopv = jnp.arange(N, dtype=jnp.int32)
    pad = EPAD - ET
    srcv = jnp.concatenate([edge_index[0].astype(jnp.int32), loopv,
                            jnp.zeros((pad,), jnp.int32)])
    dstv = jnp.concatenate([edge_index[1].astype(jnp.int32), loopv,
                            jnp.full((pad,), GARBAGE_ROW, jnp.int32)])
    gsrc = srcv[None, :] + (jnp.arange(PAIRS, dtype=jnp.int32) * N)[:, None]
    x_tab = x_seq.reshape(PAIRS, 2, N, F)
    zeros_blk = jnp.zeros((AGGR // 16, 2 * H), jnp.bfloat16)

    deg_parts = _deg(dstv)
    dinv = _dinv(deg_parts)
    table = _table(x_tab, W_gcn, dinv)
    agg = _seg(table, gsrc, dstv, zeros_blk)
    emb = _emb(agg, dinv, b_gcn.reshape(1, H)).reshape(K, H)
    emb_tbh = emb.reshape(B, T, H).transpose(1, 0, 2)
    return _head(emb_tbh, W_ih, W_hh, b_ih.reshape(1, 4 * H),
                 b_hh.reshape(1, 4 * H), W_cls, b_cls.reshape(1, 1))


# speedup vs baseline: 1.2471x; 1.2471x over previous
import functools

import jax
import jax.numpy as jnp
from jax import lax
from jax.experimental import pallas as pl
from jax.experimental.pallas import tpu as pltpu
from jax.experimental.pallas import tpu_sc as plsc

N = 10000
NPAD = 10240
E = 320000
F = 128
H = 128
B = 2
T = 8
K = B * T
ET = E + N
TR = 164
EPAD = TR * 128 * 16
DEG_CHUNK = EPAD // 32
NB = 5
NBK = N // NB
GARBAGE_ROW = N
AGGR = 10016
PAIRS = K // 2

_sc_mesh = plsc.VectorSubcoreMesh(
    core_axis_name="c", subcore_axis_name="s", num_cores=2, num_subcores=16)


@functools.partial(
    pl.kernel,
    out_type=jax.ShapeDtypeStruct((32, NPAD), jnp.float32),
    mesh=_sc_mesh,
    compiler_params=pltpu.CompilerParams(needs_layout_passes=False, use_tc_tiling_on_sc=False),
    scratch_types=[
        pltpu.VMEM((DEG_CHUNK,), jnp.int32),
        pltpu.VMEM((NPAD,), jnp.float32),
    ])
def _deg(dstv_hbm, out_hbm, dloc, degv):
    c = lax.axis_index("c")
    s = lax.axis_index("s")
    wid = s * 2 + c
    pltpu.sync_copy(dstv_hbm.at[pl.ds(wid * DEG_CHUNK, DEG_CHUNK)], dloc)

    def zbody(i, carry):
        degv[pl.ds(i * 16, 16)] = jnp.zeros((16,), jnp.float32)
        return carry

    lax.fori_loop(0, NPAD // 16, zbody, 0)
    ones = jnp.ones((16,), jnp.float32)

    def sbody(i, carry):
        idx = dloc[pl.ds(i * 16, 16)]
        plsc.addupdate_scatter(degv, [idx], ones)
        return carry

    lax.fori_loop(0, DEG_CHUNK // 16, sbody, 0)
    pltpu.sync_copy(degv, out_hbm.at[wid])


def _dinv_body(parts_ref, out_ref):
    parts = parts_ref[...]
    ones = jnp.ones((32, 1), jnp.float32)
    tot = lax.dot_general(parts, ones, (((0,), (0,)), ((), ())),
                          preferred_element_type=jnp.float32)
    out_ref[...] = lax.rsqrt(tot)


def _dinv(parts):
    return pl.pallas_call(
        _dinv_body,
        out_shape=jax.ShapeDtypeStruct((NPAD, 1), jnp.float32),
    )(parts)


def _table_body(x_ref, w_ref, dinv_ref, out_ref):
    w = w_ref[...]
    d = dinv_ref[...]
    h0 = jnp.dot(x_ref[0, 0], w, preferred_element_type=jnp.float32)
    h1 = jnp.dot(x_ref[0, 1], w, preferred_element_type=jnp.float32)
    out_ref[...] = jnp.concatenate([h0 * d, h1 * d],
                                   axis=1).astype(jnp.bfloat16)


def _table(x_tab, W_gcn, dinv):
    return pl.pallas_call(
        _table_body,
        grid=(PAIRS, NB),
        in_specs=[
            pl.BlockSpec((1, 2, NBK, F), lambda p, nb: (p, 0, nb, 0)),
            pl.BlockSpec((F, H), lambda p, nb: (0, 0)),
            pl.BlockSpec((NBK, 1), lambda p, nb: (nb, 0)),
        ],
        out_specs=pl.BlockSpec((NBK, 2 * H), lambda p, nb: (p * NB + nb, 0)),
        out_shape=jax.ShapeDtypeStruct((PAIRS * N, 2 * H), jnp.bfloat16),
    )(x_tab, W_gcn, dinv)


@functools.partial(
    pl.kernel,
    out_type=jax.ShapeDtypeStruct((PAIRS, N, 2 * H), jnp.bfloat16),
    mesh=_sc_mesh,
    compiler_params=pltpu.CompilerParams(needs_layout_passes=False, use_tc_tiling_on_sc=False),
    scratch_types=(
        [pltpu.VMEM((128,), jnp.int32) for _ in range(4)]
        + [pltpu.VMEM((128,), jnp.int32) for _ in range(4)]
        + [pltpu.VMEM((128, 2 * H), jnp.bfloat16) for _ in range(2)]
        + [pltpu.VMEM_SHARED((AGGR, 2 * H), jnp.bfloat16)]
        + [pltpu.SemaphoreType.DMA for _ in range(10)]
    ))
def _seg(table_hbm, gsrc_hbm, dstv_hbm, zeros_hbm, agg_hbm,
         gb0, gb1, gb2, gb3, db0, db1, db2, db3, r0, r1, aggsh,
         si0, si1, si2, si3, sj0, sj1, sj2, sj3, sg0, sg1):
    c = lax.axis_index("c")
    s = lax.axis_index("s")
    base = s * (TR * 128)
    gb = [gb0, gb1, gb2, gb3]
    db = [db0, db1, db2, db3]
    rr = [r0, r1]
    si = [si0, si1, si2, si3]
    sj = [sj0, sj1, sj2, sj3]
    sg = [sg0, sg1]

    def fire_idx(b, j, pr):
        pltpu.async_copy(gsrc_hbm.at[pr, pl.ds(base + j * 128, 128)],
                         gb[b], si[b])
        pltpu.async_copy(dstv_hbm.at[pl.ds(base + j * 128, 128)],
                         db[b], sj[b])

    def wait_idx(b, pr):
        pltpu.make_async_copy(gsrc_hbm.at[pr, pl.ds(base, 128)],
                              gb[b], si[b]).wait()
        pltpu.make_async_copy(dstv_hbm.at[pl.ds(base, 128)],
                              db[b], sj[b]).wait()

    def fire_gather(b, rb):
        pltpu.async_copy(table_hbm.at[gb[b]], rr[rb], sg[rb])

    def wait_gather(b, rb):
        pltpu.make_async_copy(table_hbm.at[gb[b]], rr[rb], sg[rb]).wait()

    def scatter(b, rb):
        pltpu.sync_copy(rr[rb], aggsh.at[db[b]], add=True)

    for p in range(PAIRS // 2):
        pr = c + 2 * p
        pltpu.sync_copy(zeros_hbm, aggsh.at[pl.ds(s * (AGGR // 16),
                                                  AGGR // 16)])
        for b in range(4):
            fire_idx(b, b, pr)
        wait_idx(0, pr)
        fire_gather(0, 0)
        wait_idx(1, pr)
        fire_gather(1, 1)
        plsc.subcore_barrier()

        def body(i, carry):
            jo = 4 * i
            for b in range(4):
                rb = b % 2
                wait_gather(b, rb)
                scatter(b, rb)
                fire_idx(b, jo + b + 4, pr)
                b2 = (b + 2) % 4
                wait_idx(b2, pr)
                fire_gather(b2, rb)
            return carry

        lax.fori_loop(0, TR // 4 - 1, body, 0)
        wait_gather(0, 0)
        scatter(0, 0)
        wait_idx(2, pr)
        fire_gather(2, 0)
        wait_gather(1, 1)
        scatter(1, 1)
        wait_idx(3, pr)
        fire_gather(3, 1)
        wait_gather(2, 0)
        scatter(2, 0)
        wait_gather(3, 1)
        scatter(3, 1)
        plsc.subcore_barrier()
        pltpu.sync_copy(aggsh.at[pl.ds(s * (N // 16), N // 16)],
                        agg_hbm.at[pr, pl.ds(s * (N // 16), N // 16)])
        plsc.subcore_barrier()


def _emb_body(agg_ref, dinv_ref, bg_ref, out_ref):
    nb = pl.program_id(1)
    blk = agg_ref[0].astype(jnp.float32)
    d = dinv_ref[...]
    bg = bg_ref[...]
    a0 = jnp.maximum(blk[:, :H] * d + bg, 0.0)
    a1 = jnp.maximum(blk[:, H:] * d + bg, 0.0)
    p0 = jnp.sum(a0, axis=0, keepdims=True)
    p1 = jnp.sum(a1, axis=0, keepdims=True)
    part = jnp.concatenate([p0, p1], axis=0).reshape(1, 2, H) * (1.0 / N)

    @pl.when(nb == 0)
    def _init():
        out_ref[...] = part

    @pl.when(nb > 0)
    def _acc():
        out_ref[...] = out_ref[...] + part


def _emb(agg, dinv, bg):
    return pl.pallas_call(
        _emb_body,
        grid=(PAIRS, NB),
        in_specs=[
            pl.BlockSpec((1, NBK, 2 * H), lambda p, nb: (p, nb, 0)),
            pl.BlockSpec((NBK, 1), lambda p, nb: (nb, 0)),
            pl.BlockSpec((1, H), lambda p, nb: (0, 0)),
        ],
        out_specs=pl.BlockSpec((1, 2, H), lambda p, nb: (p, 0, 0)),
        out_shape=jax.ShapeDtypeStruct((PAIRS, 2, H), jnp.float32),
    )(agg, dinv, bg)


def _head_body(emb_ref, wih_ref, whh_ref, bih_ref, bhh_ref, wcls_ref,
               bcls_ref, out_ref):
    wih = wih_ref[...]
    whh = whh_ref[...]
    b = bih_ref[...] + bhh_ref[...]

    def step(t, hc):
        h, cc = hc
        x_t = emb_ref[t]
        gates = (lax.dot_general(x_t, wih, (((1,), (1,)), ((), ())),
                                 preferred_element_type=jnp.float32)
                 + lax.dot_general(h, whh, (((1,), (1,)), ((), ())),
                                   preferred_element_type=jnp.float32)
                 + b)
        i = jax.nn.sigmoid(gates[:, 0 * H:1 * H])
        f = jax.nn.sigmoid(gates[:, 1 * H:2 * H])
        g = jnp.tanh(gates[:, 2 * H:3 * H])
        o = jax.nn.sigmoid(gates[:, 3 * H:4 * H])
        c_new = f * cc + i * g
        h_new = o * jnp.tanh(c_new)
        return (h_new, c_new)

    h0 = jnp.zeros((B, H), jnp.float32)
    c0 = jnp.zeros((B, H), jnp.float32)
    h, _ = lax.fori_loop(0, T, step, (h0, c0))
    out_ref[...] = jax.nn.sigmoid(
        jnp.dot(h, wcls_ref[...], preferred_element_type=jnp.float32)
        + bcls_ref[...])


def _head(emb_tbh, W_ih, W_hh, b_ih, b_hh, W_cls, b_cls):
    return pl.pallas_call(
        _head_body,
        out_shape=jax.ShapeDtypeStruct((B, 1), jnp.float32),
    )(emb_tbh, W_ih, W_hh, b_ih, b_hh, W_cls, b_cls)


def kernel(x_seq, edge_index, W_gcn, b_gcn, W_ih, W_hh, b_ih, b_hh, W_cls,
           b_cls):
    loopv = jnp.arange(N, dtype=jnp.int32)
    pad = EPAD - ET
    srcv = jnp.concatenate([edge_index[0].astype(jnp.int32), loopv,
                            jnp.zeros((pad,), jnp.int32)])
    dstv = jnp.concatenate([edge_index[1].astype(jnp.int32), loopv,
                            jnp.full((pad,), GARBAGE_ROW, jnp.int32)])
    gsrc = srcv[None, :] + (jnp.arange(PAIRS, dtype=jnp.int32) * N)[:, None]
    x_tab = x_seq.reshape(PAIRS, 2, N, F)
    zeros_blk = jnp.zeros((AGGR // 16, 2 * H), jnp.bfloat16)

    deg_parts = _deg(dstv)
    dinv = _dinv(deg_parts)
    table = _table(x_tab, W_gcn, dinv)
    agg = _seg(table, gsrc, dstv, zeros_blk)
    emb = _emb(agg, dinv, b_gcn.reshape(1, H)).reshape(K, H)
    emb_tbh = emb.reshape(B, T, H).transpose(1, 0, 2)
    return _head(emb_tbh, W_ih, W_hh, b_ih.reshape(1, 4 * H),
                 b_hh.reshape(1, 4 * H), W_cls, b_cls.reshape(1, 1))


# speedup vs baseline: 1.2660x; 1.0151x over previous
import functools

import jax
import jax.numpy as jnp
from jax import lax
from jax.experimental import pallas as pl
from jax.experimental.pallas import tpu as pltpu
from jax.experimental.pallas import tpu_sc as plsc

N = 10000
NPAD = 10240
E = 320000
F = 128
H = 128
B = 2
T = 8
K = B * T
ET = E + N
TR = 164
EPAD = TR * 128 * 16
DEG_CHUNK = EPAD // 32
NB = 5
NBK = N // NB
GARBAGE_ROW = N
AGGR = 10016
PAIRS = K // 2

_sc_mesh = plsc.VectorSubcoreMesh(
    core_axis_name="c", subcore_axis_name="s", num_cores=2, num_subcores=16)


@functools.partial(
    pl.kernel,
    out_type=jax.ShapeDtypeStruct((32, NPAD), jnp.float32),
    mesh=_sc_mesh,
    compiler_params=pltpu.CompilerParams(needs_layout_passes=False, use_tc_tiling_on_sc=False),
    scratch_types=[
        pltpu.VMEM((DEG_CHUNK,), jnp.int32),
        pltpu.VMEM((NPAD,), jnp.float32),
    ])
def _deg(dstv_hbm, out_hbm, dloc, degv):
    c = lax.axis_index("c")
    s = lax.axis_index("s")
    wid = s * 2 + c
    pltpu.sync_copy(dstv_hbm.at[pl.ds(wid * DEG_CHUNK, DEG_CHUNK)], dloc)

    def zbody(i, carry):
        degv[pl.ds(i * 16, 16)] = jnp.zeros((16,), jnp.float32)
        return carry

    lax.fori_loop(0, NPAD // 16, zbody, 0)
    ones = jnp.ones((16,), jnp.float32)

    def sbody(i, carry):
        idx = dloc[pl.ds(i * 16, 16)]
        plsc.addupdate_scatter(degv, [idx], ones)
        return carry

    lax.fori_loop(0, DEG_CHUNK // 16, sbody, 0)
    pltpu.sync_copy(degv, out_hbm.at[wid])


def _dinv_body(parts_ref, out_ref):
    parts = parts_ref[...]
    ones = jnp.ones((32, 1), jnp.float32)
    tot = lax.dot_general(parts, ones, (((0,), (0,)), ((), ())),
                          preferred_element_type=jnp.float32)
    out_ref[...] = lax.rsqrt(tot)


def _dinv(parts):
    return pl.pallas_call(
        _dinv_body,
        out_shape=jax.ShapeDtypeStruct((NPAD, 1), jnp.float32),
    )(parts)


def _table_body(x_ref, w_ref, dinv_ref, out_ref):
    w = w_ref[...]
    d = dinv_ref[...]
    h0 = jnp.dot(x_ref[0, 0], w, preferred_element_type=jnp.float32)
    h1 = jnp.dot(x_ref[0, 1], w, preferred_element_type=jnp.float32)
    out_ref[...] = jnp.concatenate([h0 * d, h1 * d],
                                   axis=1).astype(jnp.bfloat16)


def _table(x_tab, W_gcn, dinv):
    return pl.pallas_call(
        _table_body,
        grid=(PAIRS, NB),
        in_specs=[
            pl.BlockSpec((1, 2, NBK, F), lambda p, nb: (p, 0, nb, 0)),
            pl.BlockSpec((F, H), lambda p, nb: (0, 0)),
            pl.BlockSpec((NBK, 1), lambda p, nb: (nb, 0)),
        ],
        out_specs=pl.BlockSpec((NBK, 2 * H), lambda p, nb: (p * NB + nb, 0)),
        out_shape=jax.ShapeDtypeStruct((PAIRS * N, 2 * H), jnp.bfloat16),
    )(x_tab, W_gcn, dinv)


@functools.partial(
    pl.kernel,
    out_type=jax.ShapeDtypeStruct((PAIRS, N, 2 * H), jnp.bfloat16),
    mesh=_sc_mesh,
    compiler_params=pltpu.CompilerParams(needs_layout_passes=False, use_tc_tiling_on_sc=False),
    scratch_types=(
        [pltpu.VMEM((128,), jnp.int32) for _ in range(4)]
        + [pltpu.VMEM((128,), jnp.int32) for _ in range(4)]
        + [pltpu.VMEM((128, 2 * H), jnp.bfloat16) for _ in range(2)]
        + [pltpu.VMEM_SHARED((AGGR, 2 * H), jnp.bfloat16)]
        + [pltpu.SemaphoreType.DMA for _ in range(10)]
    ))
def _seg(table_hbm, srcv_hbm, dstv_hbm, zeros_hbm, agg_hbm,
         gb0, gb1, gb2, gb3, db0, db1, db2, db3, r0, r1, aggsh,
         si0, si1, si2, si3, sj0, sj1, sj2, sj3, sg0, sg1):
    c = lax.axis_index("c")
    s = lax.axis_index("s")
    base = s * (TR * 128)
    gb = [gb0, gb1, gb2, gb3]
    db = [db0, db1, db2, db3]
    rr = [r0, r1]
    si = [si0, si1, si2, si3]
    sj = [sj0, sj1, sj2, sj3]
    sg = [sg0, sg1]

    def fire_idx(b, j, pr):
        pltpu.async_copy(srcv_hbm.at[pl.ds(base + j * 128, 128)],
                         gb[b], si[b])
        pltpu.async_copy(dstv_hbm.at[pl.ds(base + j * 128, 128)],
                         db[b], sj[b])

    def wait_idx(b, pr):
        pltpu.make_async_copy(srcv_hbm.at[pl.ds(base, 128)],
                              gb[b], si[b]).wait()
        pltpu.make_async_copy(dstv_hbm.at[pl.ds(base, 128)],
                              db[b], sj[b]).wait()
        prn = pr * N
        for l in range(8):
            gb[b][pl.ds(l * 16, 16)] = gb[b][pl.ds(l * 16, 16)] + prn

    def fire_gather(b, rb):
        pltpu.async_copy(table_hbm.at[gb[b]], rr[rb], sg[rb])

    def wait_gather(b, rb):
        pltpu.make_async_copy(table_hbm.at[gb[b]], rr[rb], sg[rb]).wait()

    def scatter(b, rb):
        pltpu.sync_copy(rr[rb], aggsh.at[db[b]], add=True)

    for p in range(PAIRS // 2):
        pr = c + 2 * p
        pltpu.sync_copy(zeros_hbm, aggsh.at[pl.ds(s * (AGGR // 16),
                                                  AGGR // 16)])
        for b in range(4):
            fire_idx(b, b, pr)
        wait_idx(0, pr)
        fire_gather(0, 0)
        wait_idx(1, pr)
        fire_gather(1, 1)
        plsc.subcore_barrier()

        def body(i, carry):
            jo = 4 * i
            for b in range(4):
                rb = b % 2
                wait_gather(b, rb)
                scatter(b, rb)
                fire_idx(b, jo + b + 4, pr)
                b2 = (b + 2) % 4
                wait_idx(b2, pr)
                fire_gather(b2, rb)
            return carry

        lax.fori_loop(0, TR // 4 - 1, body, 0)
        wait_gather(0, 0)
        scatter(0, 0)
        wait_idx(2, pr)
        fire_gather(2, 0)
        wait_gather(1, 1)
        scatter(1, 1)
        wait_idx(3, pr)
        fire_gather(3, 1)
        wait_gather(2, 0)
        scatter(2, 0)
        wait_gather(3, 1)
        scatter(3, 1)
        plsc.subcore_barrier()
        pltpu.sync_copy(aggsh.at[pl.ds(s * (N // 16), N // 16)],
                        agg_hbm.at[pr, pl.ds(s * (N // 16), N // 16)])
        plsc.subcore_barrier()


def _emb_body(agg_ref, dinv_ref, bg_ref, out_ref):
    nb = pl.program_id(1)
    blk = agg_ref[0].astype(jnp.float32)
    d = dinv_ref[...]
    bg = bg_ref[...]
    a0 = jnp.maximum(blk[:, :H] * d + bg, 0.0)
    a1 = jnp.maximum(blk[:, H:] * d + bg, 0.0)
    p0 = jnp.sum(a0, axis=0, keepdims=True)
    p1 = jnp.sum(a1, axis=0, keepdims=True)
    part = jnp.concatenate([p0, p1], axis=0).reshape(1, 2, H) * (1.0 / N)

    @pl.when(nb == 0)
    def _init():
        out_ref[...] = part

    @pl.when(nb > 0)
    def _acc():
        out_ref[...] = out_ref[...] + part


def _emb(agg, dinv, bg):
    return pl.pallas_call(
        _emb_body,
        grid=(PAIRS, NB),
        in_specs=[
            pl.BlockSpec((1, NBK, 2 * H), lambda p, nb: (p, nb, 0)),
            pl.BlockSpec((NBK, 1), lambda p, nb: (nb, 0)),
            pl.BlockSpec((1, H), lambda p, nb: (0, 0)),
        ],
        out_specs=pl.BlockSpec((1, 2, H), lambda p, nb: (p, 0, 0)),
        out_shape=jax.ShapeDtypeStruct((PAIRS, 2, H), jnp.float32),
    )(agg, dinv, bg)


def _head_body(emb_ref, wih_ref, whh_ref, bih_ref, bhh_ref, wcls_ref,
               bcls_ref, out_ref):
    wih = wih_ref[...]
    whh = whh_ref[...]
    b = bih_ref[...] + bhh_ref[...]

    def step(t, hc):
        h, cc = hc
        x_t = emb_ref[t]
        gates = (lax.dot_general(x_t, wih, (((1,), (1,)), ((), ())),
                                 preferred_element_type=jnp.float32)
                 + lax.dot_general(h, whh, (((1,), (1,)), ((), ())),
                                   preferred_element_type=jnp.float32)
                 + b)
        i = jax.nn.sigmoid(gates[:, 0 * H:1 * H])
        f = jax.nn.sigmoid(gates[:, 1 * H:2 * H])
        g = jnp.tanh(gates[:, 2 * H:3 * H])
        o = jax.nn.sigmoid(gates[:, 3 * H:4 * H])
        c_new = f * cc + i * g
        h_new = o * jnp.tanh(c_new)
        return (h_new, c_new)

    h0 = jnp.zeros((B, H), jnp.float32)
    c0 = jnp.zeros((B, H), jnp.float32)
    h, _ = lax.fori_loop(0, T, step, (h0, c0))
    out_ref[...] = jax.nn.sigmoid(
        jnp.dot(h, wcls_ref[...], preferred_element_type=jnp.float32)
        + bcls_ref[...])


def _head(emb_tbh, W_ih, W_hh, b_ih, b_hh, W_cls, b_cls):
    return pl.pallas_call(
        _head_body,
        out_shape=jax.ShapeDtypeStruct((B, 1), jnp.float32),
    )(emb_tbh, W_ih, W_hh, b_ih, b_hh, W_cls, b_cls)


def kernel(x_seq, edge_index, W_gcn, b_gcn, W_ih, W_hh, b_ih, b_hh, W_cls,
           b_cls):
    loopv = jnp.arange(N, dtype=jnp.int32)
    pad = EPAD - ET
    srcv = jnp.concatenate([edge_index[0].astype(jnp.int32), loopv,
                            jnp.zeros((pad,), jnp.int32)])
    dstv = jnp.concatenate([edge_index[1].astype(jnp.int32), loopv,
                            jnp.full((pad,), GARBAGE_ROW, jnp.int32)])
    x_tab = x_seq.reshape(PAIRS, 2, N, F)
    zeros_blk = jnp.zeros((AGGR // 16, 2 * H), jnp.bfloat16)

    deg_parts = _deg(dstv)
    dinv = _dinv(deg_parts)
    table = _table(x_tab, W_gcn, dinv)
    agg = _seg(table, srcv, dstv, zeros_blk)
    emb = _emb(agg, dinv, b_gcn.reshape(1, H)).reshape(K, H)
    emb_tbh = emb.reshape(B, T, H).transpose(1, 0, 2)
    return _head(emb_tbh, W_ih, W_hh, b_ih.reshape(1, 4 * H),
                 b_hh.reshape(1, 4 * H), W_cls, b_cls.reshape(1, 1))
